# baseline (speedup 1.0000x reference)
import functools

import jax
import jax.numpy as jnp
from jax import lax
from jax.experimental import pallas as pl
from jax.experimental.pallas import tpu as pltpu
from jax.experimental.pallas import tpu_sc as plsc

B, N, C, G = 16, 20000, 80, 32
CH = 1280
K = -(-N // CH)


def _iou_argmax_body(bpT_ref, gt_ref, gt0_ref, idx_ref, bxp_ref,
                     mx_ref, ai_ref, bx_ref):
    b = pl.program_id(0)
    k = pl.program_id(1)

    @pl.when(k == 0)
    def _():
        mx_ref[...] = jnp.full((G, 1), -jnp.inf, jnp.float32)
        ai_ref[...] = jnp.zeros((G, 1), jnp.int32)
        bx_ref[...] = jnp.zeros((G, 4), jnp.float32)

    bp = jnp.transpose(bpT_ref[0], (1, 0))
    x1p = bp[0:1, :]
    y1p = bp[1:2, :]
    x2p = bp[2:3, :]
    y2p = bp[3:4, :]
    area_p = (x2p - x1p) * (y2p - y1p)

    gt = gt_ref[0]
    gx1 = gt[:, 0:1]
    gy1 = gt[:, 1:2]
    gx2 = gt[:, 2:3]
    gy2 = gt[:, 3:4]
    area_g = (gx2 - gx1) * (gy2 - gy1)

    w = jnp.maximum(jnp.minimum(gx2, x2p) - jnp.maximum(gx1, x1p), 0.0)
    h = jnp.maximum(jnp.minimum(gy2, y2p) - jnp.maximum(gy1, y1p), 0.0)
    inter = w * h
    union = (area_g + area_p) - inter
    lane = lax.broadcasted_iota(jnp.int32, (G, CH), 1)
    gidx = lane + k * CH
    iou = jnp.where(gidx < N, inter / union, -jnp.inf)

    m = jnp.max(iou, axis=1, keepdims=True)
    aidx = jnp.min(jnp.where(iou == m, gidx, N), axis=1, keepdims=True)
    one = gidx == aidx
    bx1 = jnp.sum(jnp.where(one, x1p, 0.0), axis=1, keepdims=True)
    by1 = jnp.sum(jnp.where(one, y1p, 0.0), axis=1, keepdims=True)
    bx2 = jnp.sum(jnp.where(one, x2p, 0.0), axis=1, keepdims=True)
    by2 = jnp.sum(jnp.where(one, y2p, 0.0), axis=1, keepdims=True)
    nbox = jnp.concatenate([bx1, by1, bx2, by2], axis=1)

    upd = m > mx_ref[...]
    mx_ref[...] = jnp.where(upd, m, mx_ref[...])
    ai_ref[...] = jnp.where(upd, aidx, ai_ref[...])
    bx_ref[...] = jnp.where(upd, nbox, bx_ref[...])

    idx_ref[0] = ai_ref[...] + b * N

    tgt = gt0_ref[0, pl.ds(b, 1), :]
    d = bx_ref[...] - tgt
    ad = jnp.abs(d)
    sl1 = jnp.where(ad < 1.0, 0.5 * d * d, ad - 0.5)
    bxp_ref[...] = jnp.reshape(jnp.sum(sl1), (1, 1, 1))


def _loss_body(x_ref, lbl_ref, bxp_ref, out_ref):
    x = x_ref[...]
    lbl = lbl_ref[...]
    iota = lax.broadcasted_iota(jnp.int32, (B * G, C), 1)
    z = (iota == jnp.clip(lbl, 0, C - 1)).astype(jnp.float32)
    bce = jnp.maximum(x, 0.0) - x * z + jnp.log(1.0 + jnp.exp(-jnp.abs(x)))
    total = jnp.sum(bce) / (B * G * C) + jnp.sum(bxp_ref[...]) / (B * G * 4)
    out_ref[...] = jnp.reshape(total, (1, 1))


_NC, _NS = 2, 16
_NW = _NC * _NS
_RPW = (B * G) // _NW


@functools.cache
def _make_sc_gather():
    @functools.partial(
        pl.kernel,
        out_type=jax.ShapeDtypeStruct((B * G, C), jnp.float32),
        mesh=plsc.VectorSubcoreMesh(core_axis_name="c", subcore_axis_name="s"),
        scratch_types=[
            pltpu.VMEM((_RPW,), jnp.int32),
            pltpu.VMEM((_RPW, C), jnp.float32),
            pltpu.SemaphoreType.DMA,
        ],
        compiler_params=pltpu.CompilerParams(use_tc_tiling_on_sc=False),
    )
    def _sc_gather(table_hbm, idx_hbm, out_hbm, idx_v, rows_v, sem):
        wid = lax.axis_index("s") * _NC + lax.axis_index("c")
        base = wid * _RPW
        pltpu.sync_copy(idx_hbm.at[pl.ds(base, _RPW)], idx_v)
        pltpu.async_copy(table_hbm.at[idx_v], rows_v, sem).wait()
        pltpu.sync_copy(rows_v, out_hbm.at[pl.ds(base, _RPW)])

    return _sc_gather


def _stage_a(bpT, gt_boxes, interpret=False):
    return pl.pallas_call(
        _iou_argmax_body,
        grid=(B, K),
        in_specs=[
            pl.BlockSpec((1, CH, 4), lambda b, k: (b, k, 0)),
            pl.BlockSpec((1, G, 4), lambda b, k: (b, 0, 0)),
            pl.BlockSpec((1, G, 4), lambda b, k: (0, 0, 0)),
        ],
        out_specs=[
            pl.BlockSpec((1, G, 1), lambda b, k: (b, 0, 0)),
            pl.BlockSpec((1, 1, 1), lambda b, k: (b, 0, 0)),
        ],
        out_shape=[
            jax.ShapeDtypeStruct((B, G, 1), jnp.int32),
            jax.ShapeDtypeStruct((B, 1, 1), jnp.float32),
        ],
        scratch_shapes=[
            pltpu.VMEM((G, 1), jnp.float32),
            pltpu.VMEM((G, 1), jnp.int32),
            pltpu.VMEM((G, 4), jnp.float32),
        ],
        compiler_params=pltpu.CompilerParams(
            dimension_semantics=("arbitrary", "arbitrary")),
        interpret=interpret,
    )(bpT, gt_boxes, gt_boxes)


def _stage_c(gathered, lbl2, bxp, interpret=False):
    return pl.pallas_call(
        _loss_body,
        out_shape=jax.ShapeDtypeStruct((1, 1), jnp.float32),
        interpret=interpret,
    )(gathered, lbl2, bxp)


def kernel(cls_preds, box_preds, gt_boxes, gt_labels):
    flat_idx3, bxp = _stage_a(box_preds, gt_boxes)
    flat_idx = flat_idx3.reshape(B * G)
    gathered = _make_sc_gather()(cls_preds.reshape(B * N, C), flat_idx)
    lbl2 = gt_labels.reshape(B * G, 1)
    out = _stage_c(gathered, lbl2, bxp)
    return out.reshape(())


# speedup vs baseline: 1.3988x; 1.3988x over previous
import functools

import jax
import jax.numpy as jnp
from jax import lax
from jax.experimental import pallas as pl
from jax.experimental.pallas import tpu as pltpu
from jax.experimental.pallas import tpu_sc as plsc

B, N, C, G = 16, 20000, 80, 32
CH = 1280
K = -(-N // CH)


def _iou_argmax_body(bpT_ref, gt_ref, gt0_ref, idx_ref, bxp_ref,
                     mx_ref, ai_ref, bx_ref):
    b = pl.program_id(0)
    k = pl.program_id(1)

    @pl.when(k == 0)
    def _():
        mx_ref[...] = jnp.full((G, 1), -jnp.inf, jnp.float32)
        ai_ref[...] = jnp.zeros((G, 1), jnp.int32)
        bx_ref[...] = jnp.zeros((G, 4), jnp.float32)

    bp = jnp.transpose(bpT_ref[0], (1, 0))
    x1p = bp[0:1, :]
    y1p = bp[1:2, :]
    x2p = bp[2:3, :]
    y2p = bp[3:4, :]
    area_p = (x2p - x1p) * (y2p - y1p)

    gt = gt_ref[0]
    gx1 = gt[:, 0:1]
    gy1 = gt[:, 1:2]
    gx2 = gt[:, 2:3]
    gy2 = gt[:, 3:4]
    area_g = (gx2 - gx1) * (gy2 - gy1)

    w = jnp.maximum(jnp.minimum(gx2, x2p) - jnp.maximum(gx1, x1p), 0.0)
    h = jnp.maximum(jnp.minimum(gy2, y2p) - jnp.maximum(gy1, y1p), 0.0)
    inter = w * h
    union = (area_g + area_p) - inter
    lane = lax.broadcasted_iota(jnp.int32, (G, CH), 1)
    gidx = lane + k * CH
    iou = jnp.where(gidx < N, inter / union, -jnp.inf)

    m = jnp.max(iou, axis=1, keepdims=True)
    aidx = jnp.min(jnp.where(iou == m, gidx, N), axis=1, keepdims=True)
    one = gidx == aidx
    bx1 = jnp.sum(jnp.where(one, x1p, 0.0), axis=1, keepdims=True)
    by1 = jnp.sum(jnp.where(one, y1p, 0.0), axis=1, keepdims=True)
    bx2 = jnp.sum(jnp.where(one, x2p, 0.0), axis=1, keepdims=True)
    by2 = jnp.sum(jnp.where(one, y2p, 0.0), axis=1, keepdims=True)
    nbox = jnp.concatenate([bx1, by1, bx2, by2], axis=1)

    upd = m > mx_ref[...]
    mx_ref[...] = jnp.where(upd, m, mx_ref[...])
    ai_ref[...] = jnp.where(upd, aidx, ai_ref[...])
    bx_ref[...] = jnp.where(upd, nbox, bx_ref[...])

    idx_ref[0] = ai_ref[...] + b * N

    tgt = gt0_ref[0, pl.ds(b, 1), :]
    d = bx_ref[...] - tgt
    ad = jnp.abs(d)
    sl1 = jnp.where(ad < 1.0, 0.5 * d * d, ad - 0.5)
    bxp_ref[...] = jnp.reshape(jnp.sum(sl1), (1, 1, 1))


def _loss_body(idx_ref, cls_ref, lbl_ref, bxp_ref, out_ref, rows_ref, sem):
    def issue(i, _):
        r = idx_ref[i]
        pltpu.make_async_copy(
            cls_ref.at[pl.ds(r, 1), :], rows_ref.at[pl.ds(i, 1), :], sem
        ).start()
        return 0

    lax.fori_loop(0, B * G, issue, 0)

    def drain(i, _):
        r = idx_ref[i]
        pltpu.make_async_copy(
            cls_ref.at[pl.ds(r, 1), :], rows_ref.at[pl.ds(i, 1), :], sem
        ).wait()
        return 0

    lax.fori_loop(0, B * G, drain, 0)

    x = rows_ref[...]
    lbl = lbl_ref[...]
    iota = lax.broadcasted_iota(jnp.int32, (B * G, C), 1)
    z = (iota == jnp.clip(lbl, 0, C - 1)).astype(jnp.float32)
    bce = jnp.maximum(x, 0.0) - x * z + jnp.log(1.0 + jnp.exp(-jnp.abs(x)))
    total = jnp.sum(bce) / (B * G * C) + jnp.sum(bxp_ref[...]) / (B * G * 4)
    out_ref[...] = jnp.reshape(total, (1, 1))


_NC, _NS = 2, 16
_NW = _NC * _NS
_RPW = (B * G) // _NW


@functools.cache
def _make_sc_gather():
    @functools.partial(
        pl.kernel,
        out_type=jax.ShapeDtypeStruct((B * G, 8, C), jnp.float32),
        mesh=plsc.VectorSubcoreMesh(core_axis_name="c", subcore_axis_name="s"),
        scratch_types=[
            pltpu.VMEM((_RPW,), jnp.int32),
            pltpu.VMEM((_RPW, 8, C), jnp.float32),
            pltpu.SemaphoreType.DMA,
        ],
    )
    def _sc_gather(table_hbm, idx_hbm, out_hbm, idx_v, rows_v, sem):
        wid = lax.axis_index("s") * _NC + lax.axis_index("c")
        base = wid * _RPW
        pltpu.sync_copy(idx_hbm.at[pl.ds(base, _RPW)], idx_v)
        group = lax.shift_right_logical(idx_v[...], 3)
        pltpu.async_copy(table_hbm.at[group], rows_v, sem).wait()
        pltpu.sync_copy(rows_v, out_hbm.at[pl.ds(base, _RPW)])

    return _sc_gather


def _stage_a(bpT, gt_boxes, interpret=False):
    return pl.pallas_call(
        _iou_argmax_body,
        grid=(B, K),
        in_specs=[
            pl.BlockSpec((1, CH, 4), lambda b, k: (b, k, 0)),
            pl.BlockSpec((1, G, 4), lambda b, k: (b, 0, 0)),
            pl.BlockSpec((1, G, 4), lambda b, k: (0, 0, 0)),
        ],
        out_specs=[
            pl.BlockSpec((1, G, 1), lambda b, k: (b, 0, 0)),
            pl.BlockSpec((1, 1, 1), lambda b, k: (b, 0, 0)),
        ],
        out_shape=[
            jax.ShapeDtypeStruct((B, G, 1), jnp.int32),
            jax.ShapeDtypeStruct((B, 1, 1), jnp.float32),
        ],
        scratch_shapes=[
            pltpu.VMEM((G, 1), jnp.float32),
            pltpu.VMEM((G, 1), jnp.int32),
            pltpu.VMEM((G, 4), jnp.float32),
        ],
        compiler_params=pltpu.CompilerParams(
            dimension_semantics=("arbitrary", "arbitrary")),
        interpret=interpret,
    )(bpT, gt_boxes, gt_boxes)


def _stage_c(flat_idx, cls_flat, lbl2, bxp, interpret=False):
    return pl.pallas_call(
        _loss_body,
        in_specs=[
            pl.BlockSpec(memory_space=pltpu.SMEM),
            pl.BlockSpec(memory_space=pl.ANY),
            pl.BlockSpec((B * G, 1), lambda: (0, 0)),
            pl.BlockSpec((B, 1, 1), lambda: (0, 0, 0)),
        ],
        out_specs=pl.BlockSpec((1, 1), lambda: (0, 0)),
        out_shape=jax.ShapeDtypeStruct((1, 1), jnp.float32),
        scratch_shapes=[
            pltpu.VMEM((B * G, C), jnp.float32),
            pltpu.SemaphoreType.DMA,
        ],
        interpret=interpret,
    )(flat_idx, cls_flat, lbl2, bxp)


def kernel(cls_preds, box_preds, gt_boxes, gt_labels):
    flat_idx3, bxp = _stage_a(box_preds, gt_boxes)
    flat_idx = flat_idx3.reshape(B * G)
    lbl2 = gt_labels.reshape(B * G, 1)
    out = _stage_c(flat_idx, cls_preds.reshape(B * N, C), lbl2, bxp)
    return out.reshape(())


# speedup vs baseline: 2.4858x; 1.7771x over previous
import jax
import jax.numpy as jnp
from jax import lax
from jax.experimental import pallas as pl
from jax.experimental.pallas import tpu as pltpu

B, N, C, G = 16, 20000, 80, 32
CH = 1280
K = -(-N // CH)
BG = B * G
CCH = 16


def _iou_argmax_body(bpT_ref, gt_ref, gt0_ref, idx_ref, li_ref, bxp_ref,
                     mx_ref, ai_ref, bx_ref):
    b = pl.program_id(0)
    k = pl.program_id(1)

    @pl.when(k == 0)
    def _():
        mx_ref[...] = jnp.full((G, 1), -jnp.inf, jnp.float32)
        ai_ref[...] = jnp.zeros((G, 1), jnp.int32)
        bx_ref[...] = jnp.zeros((G, 4), jnp.float32)

    bp = bpT_ref[0]
    x1p = bp[0:1, :]
    y1p = bp[1:2, :]
    x2p = bp[2:3, :]
    y2p = bp[3:4, :]
    area_p = (x2p - x1p) * (y2p - y1p)

    gt = gt_ref[0]
    gx1 = gt[:, 0:1]
    gy1 = gt[:, 1:2]
    gx2 = gt[:, 2:3]
    gy2 = gt[:, 3:4]
    area_g = (gx2 - gx1) * (gy2 - gy1)

    w = jnp.maximum(jnp.minimum(gx2, x2p) - jnp.maximum(gx1, x1p), 0.0)
    h = jnp.maximum(jnp.minimum(gy2, y2p) - jnp.maximum(gy1, y1p), 0.0)
    inter = w * h
    union = (area_g + area_p) - inter
    lane = lax.broadcasted_iota(jnp.int32, (G, CH), 1)
    gidx = lane + k * CH
    iou = jnp.where(gidx < N, inter / union, -jnp.inf)

    m = jnp.max(iou, axis=1, keepdims=True)
    aidx = jnp.min(jnp.where(iou == m, gidx, N), axis=1, keepdims=True)
    one = gidx == aidx
    bx1 = jnp.sum(jnp.where(one, x1p, 0.0), axis=1, keepdims=True)
    by1 = jnp.sum(jnp.where(one, y1p, 0.0), axis=1, keepdims=True)
    bx2 = jnp.sum(jnp.where(one, x2p, 0.0), axis=1, keepdims=True)
    by2 = jnp.sum(jnp.where(one, y2p, 0.0), axis=1, keepdims=True)
    nbox = jnp.concatenate([bx1, by1, bx2, by2], axis=1)

    upd = m > mx_ref[...]
    mx_ref[...] = jnp.where(upd, m, mx_ref[...])
    ai_ref[...] = jnp.where(upd, aidx, ai_ref[...])
    bx_ref[...] = jnp.where(upd, nbox, bx_ref[...])

    idx_ref[0] = ai_ref[...] + b * N
    li_ref[0] = jnp.bitwise_and(ai_ref[...], 127)

    tgt = gt0_ref[0, pl.ds(b, 1), :]
    d = bx_ref[...] - tgt
    ad = jnp.abs(d)
    sl1 = jnp.where(ad < 1.0, 0.5 * d * d, ad - 0.5)
    bxp_ref[...] = jnp.reshape(jnp.sum(sl1), (1, 1, 1))


def _loss_body(idx_ref, cls_ref, li_ref, lbl_ref, bxp_ref, out_ref,
               cols_ref, sem):
    def _copy(i):
        r = idx_ref[i]
        b = r // N
        n = r - b * N
        blk = pl.multiple_of((n // 128) * 128, 128)
        return pltpu.make_async_copy(
            cls_ref.at[b, :, pl.ds(blk, 128)],
            cols_ref.at[:, pl.ds(pl.multiple_of(i * 128, 128), 128)],
            sem,
        )

    def issue(i, _):
        _copy(i).start()
        return 0

    lax.fori_loop(0, BG, issue, 0)

    def drain(i, _):
        _copy(i).wait()
        return 0

    lax.fori_loop(0, BG, drain, 0)

    acc = jnp.zeros((), jnp.float32)
    c3 = lax.broadcasted_iota(jnp.int32, (C, 1, 1), 0)
    lane3 = lax.broadcasted_iota(jnp.int32, (1, CCH, 128), 2)
    for j in range(BG // CCH):
        x3 = cols_ref[:, j * CCH * 128:(j + 1) * CCH * 128].reshape(
            C, CCH, 128)
        li3 = li_ref[j * CCH:(j + 1) * CCH, :].reshape(1, CCH, 1)
        lb3 = lbl_ref[j * CCH:(j + 1) * CCH, :].reshape(1, CCH, 1)
        m3 = lane3 == li3
        z3 = jnp.logical_and(m3, c3 == jnp.clip(lb3, 0, C - 1))
        xs = jnp.where(m3, x3, 0.0)
        zf = z3.astype(jnp.float32)
        bce = (jnp.maximum(xs, 0.0) - xs * zf
               + jnp.log(1.0 + jnp.exp(-jnp.abs(xs))))
        acc = acc + jnp.sum(jnp.where(m3, bce, 0.0))
    total = acc / (BG * C) + jnp.sum(bxp_ref[...]) / (BG * 4)
    out_ref[...] = jnp.reshape(total, (1, 1))


def _stage_a(bpT, gt_boxes, interpret=False):
    return pl.pallas_call(
        _iou_argmax_body,
        grid=(B, K),
        in_specs=[
            pl.BlockSpec((1, 4, CH), lambda b, k: (b, 0, k)),
            pl.BlockSpec((1, G, 4), lambda b, k: (b, 0, 0)),
            pl.BlockSpec((1, G, 4), lambda b, k: (0, 0, 0)),
        ],
        out_specs=[
            pl.BlockSpec((1, G, 1), lambda b, k: (b, 0, 0)),
            pl.BlockSpec((1, G, 1), lambda b, k: (b, 0, 0)),
            pl.BlockSpec((1, 1, 1), lambda b, k: (b, 0, 0)),
        ],
        out_shape=[
            jax.ShapeDtypeStruct((B, G, 1), jnp.int32),
            jax.ShapeDtypeStruct((B, G, 1), jnp.int32),
            jax.ShapeDtypeStruct((B, 1, 1), jnp.float32),
        ],
        scratch_shapes=[
            pltpu.VMEM((G, 1), jnp.float32),
            pltpu.VMEM((G, 1), jnp.int32),
            pltpu.VMEM((G, 4), jnp.float32),
        ],
        compiler_params=pltpu.CompilerParams(
            dimension_semantics=("arbitrary", "arbitrary")),
        interpret=interpret,
    )(bpT, gt_boxes, gt_boxes)


def _stage_c(flat_idx, cls_t, li2, lbl2, bxp, interpret=False):
    return pl.pallas_call(
        _loss_body,
        in_specs=[
            pl.BlockSpec(memory_space=pltpu.SMEM),
            pl.BlockSpec(memory_space=pl.ANY),
            pl.BlockSpec((BG, 1), lambda: (0, 0)),
            pl.BlockSpec((BG, 1), lambda: (0, 0)),
            pl.BlockSpec((B, 1, 1), lambda: (0, 0, 0)),
        ],
        out_specs=pl.BlockSpec((1, 1), lambda: (0, 0)),
        out_shape=jax.ShapeDtypeStruct((1, 1), jnp.float32),
        scratch_shapes=[
            pltpu.VMEM((C, BG * 128), jnp.float32),
            pltpu.SemaphoreType.DMA,
        ],
        interpret=interpret,
    )(flat_idx, cls_t, li2, lbl2, bxp)


def kernel(cls_preds, box_preds, gt_boxes, gt_labels):
    box_t = jnp.transpose(box_preds, (0, 2, 1))
    cls_t = jnp.transpose(cls_preds, (0, 2, 1))
    flat_idx3, li3, bxp = _stage_a(box_t, gt_boxes)
    flat_idx = flat_idx3.reshape(BG)
    out = _stage_c(flat_idx, cls_t, li3.reshape(BG, 1),
                   gt_labels.reshape(BG, 1), bxp)
    return out.reshape(())


# speedup vs baseline: 6.0205x; 2.4220x over previous
import jax
import jax.numpy as jnp
from jax import lax
from jax.experimental import pallas as pl
from jax.experimental.pallas import tpu as pltpu

B, N, C, G = 16, 20000, 80, 32
CH = 20096
K = -(-N // CH)
BG = B * G
CCH = 16


def _iou_argmax_body(bpT_ref, gt_ref, gt0_ref, idx_ref, li_ref, bxp_ref,
                     mx_ref, ai_ref, bx_ref):
    b = pl.program_id(0)
    k = pl.program_id(1)

    @pl.when(k == 0)
    def _():
        mx_ref[...] = jnp.full((G, 1), -jnp.inf, jnp.float32)
        ai_ref[...] = jnp.zeros((G, 1), jnp.int32)
        bx_ref[...] = jnp.zeros((G, 4), jnp.float32)

    bp = bpT_ref[0]
    x1p = bp[0:1, :]
    y1p = bp[1:2, :]
    x2p = bp[2:3, :]
    y2p = bp[3:4, :]
    area_p = (x2p - x1p) * (y2p - y1p)

    gt = gt_ref[0]
    gx1 = gt[:, 0:1]
    gy1 = gt[:, 1:2]
    gx2 = gt[:, 2:3]
    gy2 = gt[:, 3:4]
    area_g = (gx2 - gx1) * (gy2 - gy1)

    w = jnp.maximum(jnp.minimum(gx2, x2p) - jnp.maximum(gx1, x1p), 0.0)
    h = jnp.maximum(jnp.minimum(gy2, y2p) - jnp.maximum(gy1, y1p), 0.0)
    inter = w * h
    union = (area_g + area_p) - inter
    lane = lax.broadcasted_iota(jnp.int32, (G, CH), 1)
    gidx = lane + k * CH
    iou = jnp.where(gidx < N, inter / union, -jnp.inf)

    m = jnp.max(iou, axis=1, keepdims=True)
    aidx = jnp.min(jnp.where(iou == m, gidx, N), axis=1, keepdims=True)
    one = gidx == aidx
    bx1 = jnp.sum(jnp.where(one, x1p, 0.0), axis=1, keepdims=True)
    by1 = jnp.sum(jnp.where(one, y1p, 0.0), axis=1, keepdims=True)
    bx2 = jnp.sum(jnp.where(one, x2p, 0.0), axis=1, keepdims=True)
    by2 = jnp.sum(jnp.where(one, y2p, 0.0), axis=1, keepdims=True)
    nbox = jnp.concatenate([bx1, by1, bx2, by2], axis=1)

    upd = m > mx_ref[...]
    mx_ref[...] = jnp.where(upd, m, mx_ref[...])
    ai_ref[...] = jnp.where(upd, aidx, ai_ref[...])
    bx_ref[...] = jnp.where(upd, nbox, bx_ref[...])

    idx_ref[0] = ai_ref[...] + b * N
    li_ref[0] = jnp.bitwise_and(ai_ref[...], 127)

    tgt = gt0_ref[0, pl.ds(b, 1), :]
    d = bx_ref[...] - tgt
    ad = jnp.abs(d)
    sl1 = jnp.where(ad < 1.0, 0.5 * d * d, ad - 0.5)
    bxp_ref[...] = jnp.reshape(jnp.sum(sl1), (1, 1, 1))


def _loss_body(idx_ref, cls_ref, li_ref, lbl_ref, bxp_ref, out_ref,
               cols_ref, sem):
    def _copy(i):
        r = idx_ref[i]
        b = r // N
        n = r - b * N
        blk = pl.multiple_of((n // 128) * 128, 128)
        return pltpu.make_async_copy(
            cls_ref.at[b, :, pl.ds(blk, 128)],
            cols_ref.at[:, pl.ds(pl.multiple_of(i * 128, 128), 128)],
            sem,
        )

    def issue(i, _):
        _copy(i).start()
        return 0

    lax.fori_loop(0, BG, issue, 0)

    def drain(i, _):
        _copy(i).wait()
        return 0

    lax.fori_loop(0, BG, drain, 0)

    acc = jnp.zeros((), jnp.float32)
    c3 = lax.broadcasted_iota(jnp.int32, (C, 1, 1), 0)
    lane3 = lax.broadcasted_iota(jnp.int32, (1, CCH, 128), 2)
    for j in range(BG // CCH):
        x3 = cols_ref[:, j * CCH * 128:(j + 1) * CCH * 128].reshape(
            C, CCH, 128)
        li3 = li_ref[j * CCH:(j + 1) * CCH, :].reshape(1, CCH, 1)
        lb3 = lbl_ref[j * CCH:(j + 1) * CCH, :].reshape(1, CCH, 1)
        m3 = lane3 == li3
        z3 = jnp.logical_and(m3, c3 == jnp.clip(lb3, 0, C - 1))
        xs = jnp.where(m3, x3, 0.0)
        zf = z3.astype(jnp.float32)
        bce = (jnp.maximum(xs, 0.0) - xs * zf
               + jnp.log(1.0 + jnp.exp(-jnp.abs(xs))))
        acc = acc + jnp.sum(jnp.where(m3, bce, 0.0))
    total = acc / (BG * C) + jnp.sum(bxp_ref[...]) / (BG * 4)
    out_ref[...] = jnp.reshape(total, (1, 1))


def _stage_a(bpT, gt_boxes, interpret=False):
    return pl.pallas_call(
        _iou_argmax_body,
        grid=(B, K),
        in_specs=[
            pl.BlockSpec((1, 4, CH), lambda b, k: (b, 0, k)),
            pl.BlockSpec((1, G, 4), lambda b, k: (b, 0, 0)),
            pl.BlockSpec((1, G, 4), lambda b, k: (0, 0, 0)),
        ],
        out_specs=[
            pl.BlockSpec((1, G, 1), lambda b, k: (b, 0, 0)),
            pl.BlockSpec((1, G, 1), lambda b, k: (b, 0, 0)),
            pl.BlockSpec((1, 1, 1), lambda b, k: (b, 0, 0)),
        ],
        out_shape=[
            jax.ShapeDtypeStruct((B, G, 1), jnp.int32),
            jax.ShapeDtypeStruct((B, G, 1), jnp.int32),
            jax.ShapeDtypeStruct((B, 1, 1), jnp.float32),
        ],
        scratch_shapes=[
            pltpu.VMEM((G, 1), jnp.float32),
            pltpu.VMEM((G, 1), jnp.int32),
            pltpu.VMEM((G, 4), jnp.float32),
        ],
        compiler_params=pltpu.CompilerParams(
            dimension_semantics=("arbitrary", "arbitrary")),
        interpret=interpret,
    )(bpT, gt_boxes, gt_boxes)


def _stage_c(flat_idx, cls_t, li2, lbl2, bxp, interpret=False):
    return pl.pallas_call(
        _loss_body,
        in_specs=[
            pl.BlockSpec(memory_space=pltpu.SMEM),
            pl.BlockSpec(memory_space=pl.ANY),
            pl.BlockSpec((BG, 1), lambda: (0, 0)),
            pl.BlockSpec((BG, 1), lambda: (0, 0)),
            pl.BlockSpec((B, 1, 1), lambda: (0, 0, 0)),
        ],
        out_specs=pl.BlockSpec((1, 1), lambda: (0, 0)),
        out_shape=jax.ShapeDtypeStruct((1, 1), jnp.float32),
        scratch_shapes=[
            pltpu.VMEM((C, BG * 128), jnp.float32),
            pltpu.SemaphoreType.DMA,
        ],
        interpret=interpret,
    )(flat_idx, cls_t, li2, lbl2, bxp)


def kernel(cls_preds, box_preds, gt_boxes, gt_labels):
    box_t = jnp.transpose(box_preds, (0, 2, 1))
    cls_t = jnp.transpose(cls_preds, (0, 2, 1))
    flat_idx3, li3, bxp = _stage_a(box_t, gt_boxes)
    flat_idx = flat_idx3.reshape(BG)
    out = _stage_c(flat_idx, cls_t, li3.reshape(BG, 1),
                   gt_labels.reshape(BG, 1), bxp)
    return out.reshape(())


# speedup vs baseline: 6.7324x; 1.1182x over previous
import jax
import jax.numpy as jnp
from jax import lax
from jax.experimental import pallas as pl
from jax.experimental.pallas import tpu as pltpu

B, N, C, G = 16, 20000, 80, 32
CH = 20096
K = -(-N // CH)
BG = B * G
CCH = 16


def _iou_argmax_body(bpT_ref, gt_ref, gt0_ref, idx_ref, li_ref, bxp_ref,
                     mx_ref, ai_ref, bx_ref):
    b = pl.program_id(0)
    k = pl.program_id(1)

    @pl.when(k == 0)
    def _():
        mx_ref[...] = jnp.full((G, 1), -jnp.inf, jnp.float32)
        ai_ref[...] = jnp.zeros((G, 1), jnp.int32)
        bx_ref[...] = jnp.zeros((G, 4), jnp.float32)

    bp = bpT_ref[0]
    x1p = bp[0:1, :]
    y1p = bp[1:2, :]
    x2p = bp[2:3, :]
    y2p = bp[3:4, :]
    area_p = (x2p - x1p) * (y2p - y1p)

    gt = gt_ref[0]
    gx1 = gt[:, 0:1]
    gy1 = gt[:, 1:2]
    gx2 = gt[:, 2:3]
    gy2 = gt[:, 3:4]
    area_g = (gx2 - gx1) * (gy2 - gy1)

    w = jnp.maximum(jnp.minimum(gx2, x2p) - jnp.maximum(gx1, x1p), 0.0)
    h = jnp.maximum(jnp.minimum(gy2, y2p) - jnp.maximum(gy1, y1p), 0.0)
    inter = w * h
    union = (area_g + area_p) - inter
    lane = lax.broadcasted_iota(jnp.int32, (G, CH), 1)
    gidx = lane + k * CH
    iou = jnp.where(gidx < N, inter / union, -jnp.inf)

    m = jnp.max(iou, axis=1, keepdims=True)
    aidx = jnp.min(jnp.where(iou == m, gidx, N), axis=1, keepdims=True)
    one = (gidx == aidx).astype(jnp.float32)
    lane1 = lax.broadcasted_iota(jnp.int32, (1, CH), 1)
    bps = jnp.where(lane1 + k * CH < N, bp, 0.0)
    nbox = lax.dot_general(one, bps, (((1,), (1,)), ((), ())),
                           preferred_element_type=jnp.float32)

    upd = m > mx_ref[...]
    mx_ref[...] = jnp.where(upd, m, mx_ref[...])
    ai_ref[...] = jnp.where(upd, aidx, ai_ref[...])
    bx_ref[...] = jnp.where(upd, nbox, bx_ref[...])

    idx_ref[0] = ai_ref[...] + b * N
    li_ref[0] = jnp.bitwise_and(ai_ref[...], 127)

    tgt = gt0_ref[0, pl.ds(b, 1), :]
    d = bx_ref[...] - tgt
    ad = jnp.abs(d)
    sl1 = jnp.where(ad < 1.0, 0.5 * d * d, ad - 0.5)
    bxp_ref[...] = jnp.reshape(jnp.sum(sl1), (1, 1, 1))


def _loss_body(idx_ref, cls_ref, li_ref, lbl_ref, bxp_ref, out_ref,
               cols_ref, sem):
    def _copy(i):
        r = idx_ref[i]
        b = r // N
        n = r - b * N
        blk = pl.multiple_of((n // 128) * 128, 128)
        return pltpu.make_async_copy(
            cls_ref.at[b, :, pl.ds(blk, 128)],
            cols_ref.at[:, pl.ds(pl.multiple_of(i * 128, 128), 128)],
            sem,
        )

    def issue(i, _):
        _copy(i).start()
        return 0

    lax.fori_loop(0, BG, issue, 0)

    def drain(i, _):
        _copy(i).wait()
        return 0

    lax.fori_loop(0, BG, drain, 0)

    acc = jnp.zeros((), jnp.float32)
    c3 = lax.broadcasted_iota(jnp.int32, (C, 1, 1), 0)
    lane3 = lax.broadcasted_iota(jnp.int32, (1, CCH, 128), 2)
    for j in range(BG // CCH):
        x3 = cols_ref[:, j * CCH * 128:(j + 1) * CCH * 128].reshape(
            C, CCH, 128)
        li3 = li_ref[j * CCH:(j + 1) * CCH, :].reshape(1, CCH, 1)
        lb3 = lbl_ref[j * CCH:(j + 1) * CCH, :].reshape(1, CCH, 1)
        m3 = lane3 == li3
        z3 = jnp.logical_and(m3, c3 == jnp.clip(lb3, 0, C - 1))
        xs = jnp.where(m3, x3, 0.0)
        zf = z3.astype(jnp.float32)
        bce = (jnp.maximum(xs, 0.0) - xs * zf
               + jnp.log(1.0 + jnp.exp(-jnp.abs(xs))))
        acc = acc + jnp.sum(jnp.where(m3, bce, 0.0))
    total = acc / (BG * C) + jnp.sum(bxp_ref[...]) / (BG * 4)
    out_ref[...] = jnp.reshape(total, (1, 1))


def _stage_a(bpT, gt_boxes, interpret=False):
    return pl.pallas_call(
        _iou_argmax_body,
        grid=(B, K),
        in_specs=[
            pl.BlockSpec((1, 4, CH), lambda b, k: (b, 0, k)),
            pl.BlockSpec((1, G, 4), lambda b, k: (b, 0, 0)),
            pl.BlockSpec((1, G, 4), lambda b, k: (0, 0, 0)),
        ],
        out_specs=[
            pl.BlockSpec((1, G, 1), lambda b, k: (b, 0, 0)),
            pl.BlockSpec((1, G, 1), lambda b, k: (b, 0, 0)),
            pl.BlockSpec((1, 1, 1), lambda b, k: (b, 0, 0)),
        ],
        out_shape=[
            jax.ShapeDtypeStruct((B, G, 1), jnp.int32),
            jax.ShapeDtypeStruct((B, G, 1), jnp.int32),
            jax.ShapeDtypeStruct((B, 1, 1), jnp.float32),
        ],
        scratch_shapes=[
            pltpu.VMEM((G, 1), jnp.float32),
            pltpu.VMEM((G, 1), jnp.int32),
            pltpu.VMEM((G, 4), jnp.float32),
        ],
        compiler_params=pltpu.CompilerParams(
            dimension_semantics=("arbitrary", "arbitrary")),
        interpret=interpret,
    )(bpT, gt_boxes, gt_boxes)


def _stage_c(flat_idx, cls_t, li2, lbl2, bxp, interpret=False):
    return pl.pallas_call(
        _loss_body,
        in_specs=[
            pl.BlockSpec(memory_space=pltpu.SMEM),
            pl.BlockSpec(memory_space=pl.ANY),
            pl.BlockSpec((BG, 1), lambda: (0, 0)),
            pl.BlockSpec((BG, 1), lambda: (0, 0)),
            pl.BlockSpec((B, 1, 1), lambda: (0, 0, 0)),
        ],
        out_specs=pl.BlockSpec((1, 1), lambda: (0, 0)),
        out_shape=jax.ShapeDtypeStruct((1, 1), jnp.float32),
        scratch_shapes=[
            pltpu.VMEM((C, BG * 128), jnp.float32),
            pltpu.SemaphoreType.DMA,
        ],
        interpret=interpret,
    )(flat_idx, cls_t, li2, lbl2, bxp)


def kernel(cls_preds, box_preds, gt_boxes, gt_labels):
    box_t = jnp.transpose(box_preds, (0, 2, 1))
    cls_t = jnp.transpose(cls_preds, (0, 2, 1))
    flat_idx3, li3, bxp = _stage_a(box_t, gt_boxes)
    flat_idx = flat_idx3.reshape(BG)
    out = _stage_c(flat_idx, cls_t, li3.reshape(BG, 1),
                   gt_labels.reshape(BG, 1), bxp)
    return out.reshape(())


# speedup vs baseline: 7.2969x; 1.0839x over previous
import jax
import jax.numpy as jnp
from jax import lax
from jax.experimental import pallas as pl
from jax.experimental.pallas import tpu as pltpu

B, N, C, G = 16, 20000, 80, 32
CH = 20096
K = -(-N // CH)
BG = B * G
CCH = 16


def _iou_argmax_body(bpT_ref, gt_ref, gt0_ref, idx_ref, li_ref, bxp_ref,
                     mx_ref, ai_ref, bx_ref):
    b = pl.program_id(0)
    k = pl.program_id(1)

    @pl.when(k == 0)
    def _():
        mx_ref[...] = jnp.full((G, 1), -jnp.inf, jnp.float32)
        ai_ref[...] = jnp.zeros((G, 1), jnp.int32)
        bx_ref[...] = jnp.zeros((G, 4), jnp.float32)

    bp = bpT_ref[0]
    x1p = bp[0:1, :]
    y1p = bp[1:2, :]
    x2p = bp[2:3, :]
    y2p = bp[3:4, :]
    area_p = (x2p - x1p) * (y2p - y1p)

    gt = gt_ref[0]
    gx1 = gt[:, 0:1]
    gy1 = gt[:, 1:2]
    gx2 = gt[:, 2:3]
    gy2 = gt[:, 3:4]
    area_g = (gx2 - gx1) * (gy2 - gy1)

    w = jnp.maximum(jnp.minimum(gx2, x2p) - jnp.maximum(gx1, x1p), 0.0)
    h = jnp.maximum(jnp.minimum(gy2, y2p) - jnp.maximum(gy1, y1p), 0.0)
    inter = w * h
    union = (area_g + area_p) - inter
    lane = lax.broadcasted_iota(jnp.int32, (G, CH), 1)
    gidx = lane + k * CH
    iou = jnp.where(gidx < N, inter / union, -jnp.inf)

    m = jnp.max(iou, axis=1, keepdims=True)
    aidx = jnp.min(jnp.where(iou == m, gidx, N), axis=1, keepdims=True)
    one = (gidx == aidx).astype(jnp.float32)
    lane1 = lax.broadcasted_iota(jnp.int32, (1, CH), 1)
    bps = jnp.where(lane1 + k * CH < N, bp, 0.0)
    nbox = lax.dot_general(one, bps, (((1,), (1,)), ((), ())),
                           preferred_element_type=jnp.float32)

    upd = m > mx_ref[...]
    mx_ref[...] = jnp.where(upd, m, mx_ref[...])
    ai_ref[...] = jnp.where(upd, aidx, ai_ref[...])
    bx_ref[...] = jnp.where(upd, nbox, bx_ref[...])

    idx_ref[0] = ai_ref[...] + b * N
    li_ref[0] = jnp.bitwise_and(ai_ref[...], 127)

    tgt = gt0_ref[0, pl.ds(b, 1), :]
    d = bx_ref[...] - tgt
    ad = jnp.abs(d)
    sl1 = jnp.where(ad < 1.0, 0.5 * d * d, ad - 0.5)
    bxp_ref[...] = jnp.reshape(jnp.sum(sl1), (1, 1, 1))


def _loss_body(idx_ref, cls_ref, li_ref, lbl_ref, bxp_ref, out_ref,
               cols_ref, sem):
    def _copy(i):
        r = idx_ref[i]
        b = r // N
        n = r - b * N
        blk = pl.multiple_of((n // 128) * 128, 128)
        return pltpu.make_async_copy(
            cls_ref.at[b, :, pl.ds(blk, 128)],
            cols_ref.at[:, pl.ds(pl.multiple_of(i * 128, 128), 128)],
            sem,
        )

    def issue(i, _):
        _copy(i).start()
        return 0

    lax.fori_loop(0, BG, issue, 0)

    pltpu.make_async_copy(cols_ref, cols_ref, sem).wait()

    acc = jnp.zeros((), jnp.float32)
    c2 = lax.broadcasted_iota(jnp.int32, (C, CCH), 0)
    lane3 = lax.broadcasted_iota(jnp.int32, (1, CCH, 128), 2)
    rio = lax.broadcasted_iota(jnp.int32, (CCH * 128, CCH), 0)
    cio = lax.broadcasted_iota(jnp.int32, (CCH * 128, CCH), 1)
    T = (rio // 128 == cio).astype(jnp.float32)
    for j in range(BG // CCH):
        x3 = cols_ref[:, j * CCH * 128:(j + 1) * CCH * 128].reshape(
            C, CCH, 128)
        li3 = li_ref[j * CCH:(j + 1) * CCH, :].reshape(1, CCH, 1)
        m3 = lane3 == li3
        xs2 = jnp.where(m3, x3, 0.0).reshape(C, CCH * 128)
        x2 = lax.dot_general(xs2, T, (((1,), (0,)), ((), ())),
                             preferred_element_type=jnp.float32)
        lb2 = lbl_ref[j * CCH:(j + 1) * CCH, :].reshape(1, CCH)
        z2 = (c2 == jnp.clip(lb2, 0, C - 1)).astype(jnp.float32)
        bce = (jnp.maximum(x2, 0.0) - x2 * z2
               + jnp.log(1.0 + jnp.exp(-jnp.abs(x2))))
        acc = acc + jnp.sum(bce)
    total = acc / (BG * C) + jnp.sum(bxp_ref[...]) / (BG * 4)
    out_ref[...] = jnp.reshape(total, (1, 1))


def _stage_a(bpT, gt_boxes, interpret=False):
    return pl.pallas_call(
        _iou_argmax_body,
        grid=(B, K),
        in_specs=[
            pl.BlockSpec((1, 4, CH), lambda b, k: (b, 0, k)),
            pl.BlockSpec((1, G, 4), lambda b, k: (b, 0, 0)),
            pl.BlockSpec((1, G, 4), lambda b, k: (0, 0, 0)),
        ],
        out_specs=[
            pl.BlockSpec((1, G, 1), lambda b, k: (b, 0, 0)),
            pl.BlockSpec((1, G, 1), lambda b, k: (b, 0, 0)),
            pl.BlockSpec((1, 1, 1), lambda b, k: (b, 0, 0)),
        ],
        out_shape=[
            jax.ShapeDtypeStruct((B, G, 1), jnp.int32),
            jax.ShapeDtypeStruct((B, G, 1), jnp.int32),
            jax.ShapeDtypeStruct((B, 1, 1), jnp.float32),
        ],
        scratch_shapes=[
            pltpu.VMEM((G, 1), jnp.float32),
            pltpu.VMEM((G, 1), jnp.int32),
            pltpu.VMEM((G, 4), jnp.float32),
        ],
        compiler_params=pltpu.CompilerParams(
            dimension_semantics=("arbitrary", "arbitrary")),
        interpret=interpret,
    )(bpT, gt_boxes, gt_boxes)


def _stage_c(flat_idx, cls_t, li2, lbl2, bxp, interpret=False):
    return pl.pallas_call(
        _loss_body,
        in_specs=[
            pl.BlockSpec(memory_space=pltpu.SMEM),
            pl.BlockSpec(memory_space=pl.ANY),
            pl.BlockSpec((BG, 1), lambda: (0, 0)),
            pl.BlockSpec((BG, 1), lambda: (0, 0)),
            pl.BlockSpec((B, 1, 1), lambda: (0, 0, 0)),
        ],
        out_specs=pl.BlockSpec((1, 1), lambda: (0, 0)),
        out_shape=jax.ShapeDtypeStruct((1, 1), jnp.float32),
        scratch_shapes=[
            pltpu.VMEM((C, BG * 128), jnp.float32),
            pltpu.SemaphoreType.DMA,
        ],
        interpret=interpret,
    )(flat_idx, cls_t, li2, lbl2, bxp)


def kernel(cls_preds, box_preds, gt_boxes, gt_labels):
    box_t = jnp.transpose(box_preds, (0, 2, 1))
    cls_t = jnp.transpose(cls_preds, (0, 2, 1))
    flat_idx3, li3, bxp = _stage_a(box_t, gt_boxes)
    flat_idx = flat_idx3.reshape(BG)
    out = _stage_c(flat_idx, cls_t, li3.reshape(BG, 1),
                   gt_labels.reshape(BG, 1), bxp)
    return out.reshape(())


# speedup vs baseline: 7.3024x; 1.0007x over previous
import jax
import jax.numpy as jnp
from jax import lax
from jax.experimental import pallas as pl
from jax.experimental.pallas import tpu as pltpu

B, N, C, G = 16, 20000, 80, 32
CH = 20096
K = -(-N // CH)
BG = B * G
CCH = 16


def _iou_argmax_body(bpT_ref, gt_ref, gt0_ref, idx_ref, li_ref, bxp_ref,
                     mx_ref, ai_ref, bx_ref):
    b = pl.program_id(0)
    k = pl.program_id(1)

    @pl.when(k == 0)
    def _():
        mx_ref[...] = jnp.full((G, 1), -jnp.inf, jnp.float32)
        ai_ref[...] = jnp.zeros((G, 1), jnp.int32)
        bx_ref[...] = jnp.zeros((G, 4), jnp.float32)

    bp = bpT_ref[0]
    x1p = bp[0:1, :]
    y1p = bp[1:2, :]
    x2p = bp[2:3, :]
    y2p = bp[3:4, :]
    area_p = (x2p - x1p) * (y2p - y1p)

    gt = gt_ref[0]
    gx1 = gt[:, 0:1]
    gy1 = gt[:, 1:2]
    gx2 = gt[:, 2:3]
    gy2 = gt[:, 3:4]
    area_g = (gx2 - gx1) * (gy2 - gy1)

    w = jnp.maximum(jnp.minimum(gx2, x2p) - jnp.maximum(gx1, x1p), 0.0)
    h = jnp.maximum(jnp.minimum(gy2, y2p) - jnp.maximum(gy1, y1p), 0.0)
    inter = w * h
    union = (area_g + area_p) - inter
    lane = lax.broadcasted_iota(jnp.int32, (G, CH), 1)
    gidx = lane + k * CH
    iou = jnp.where(gidx < N, inter / union, -jnp.inf)

    m = jnp.max(iou, axis=1, keepdims=True)
    aidx = jnp.min(jnp.where(iou == m, gidx, N), axis=1, keepdims=True)
    one = (gidx == aidx).astype(jnp.float32)
    lane1 = lax.broadcasted_iota(jnp.int32, (1, CH), 1)
    bps = jnp.where(lane1 + k * CH < N, bp, 0.0)
    nbox = lax.dot_general(one, bps, (((1,), (1,)), ((), ())),
                           preferred_element_type=jnp.float32)

    upd = m > mx_ref[...]
    mx_ref[...] = jnp.where(upd, m, mx_ref[...])
    ai_ref[...] = jnp.where(upd, aidx, ai_ref[...])
    bx_ref[...] = jnp.where(upd, nbox, bx_ref[...])

    idx_ref[0] = ai_ref[...] + b * N
    li_ref[0] = jnp.bitwise_and(ai_ref[...], 127)

    tgt = gt0_ref[0, pl.ds(b, 1), :]
    d = bx_ref[...] - tgt
    ad = jnp.abs(d)
    sl1 = jnp.where(ad < 1.0, 0.5 * d * d, ad - 0.5)
    bxp_ref[...] = jnp.reshape(jnp.sum(sl1), (1, 1, 1))


def _loss_body(idx_ref, cls_ref, li_ref, lbl_ref, bxp_ref, out_ref,
               cols_ref, sem):
    def _copy(i):
        r = idx_ref[i]
        b = r // N
        n = r - b * N
        blk = pl.multiple_of((n // 128) * 128, 128)
        return pltpu.make_async_copy(
            cls_ref.at[b, :, pl.ds(blk, 128)],
            cols_ref.at[:, pl.ds(pl.multiple_of(i * 128, 128), 128)],
            sem,
        )

    def issue(i8, _):
        for u in range(8):
            _copy(i8 * 8 + u).start()
        return 0

    lax.fori_loop(0, BG // 8, issue, 0)

    pltpu.make_async_copy(cols_ref, cols_ref, sem).wait()

    acc = jnp.zeros((), jnp.float32)
    c2 = lax.broadcasted_iota(jnp.int32, (C, CCH), 0)
    lane3 = lax.broadcasted_iota(jnp.int32, (1, CCH, 128), 2)
    rio = lax.broadcasted_iota(jnp.int32, (CCH * 128, CCH), 0)
    cio = lax.broadcasted_iota(jnp.int32, (CCH * 128, CCH), 1)
    T = (rio // 128 == cio).astype(jnp.float32)
    for j in range(BG // CCH):
        x3 = cols_ref[:, j * CCH * 128:(j + 1) * CCH * 128].reshape(
            C, CCH, 128)
        li3 = li_ref[j * CCH:(j + 1) * CCH, :].reshape(1, CCH, 1)
        m3 = lane3 == li3
        xs2 = jnp.where(m3, x3, 0.0).reshape(C, CCH * 128)
        x2 = lax.dot_general(xs2, T, (((1,), (0,)), ((), ())),
                             preferred_element_type=jnp.float32)
        lb2 = lbl_ref[j * CCH:(j + 1) * CCH, :].reshape(1, CCH)
        z2 = (c2 == jnp.clip(lb2, 0, C - 1)).astype(jnp.float32)
        bce = (jnp.maximum(x2, 0.0) - x2 * z2
               + jnp.log(1.0 + jnp.exp(-jnp.abs(x2))))
        acc = acc + jnp.sum(bce)
    total = acc / (BG * C) + jnp.sum(bxp_ref[...]) / (BG * 4)
    out_ref[...] = jnp.reshape(total, (1, 1))


def _stage_a(bpT, gt_boxes, interpret=False):
    return pl.pallas_call(
        _iou_argmax_body,
        grid=(B, K),
        in_specs=[
            pl.BlockSpec((1, 4, CH), lambda b, k: (b, 0, k)),
            pl.BlockSpec((1, G, 4), lambda b, k: (b, 0, 0)),
            pl.BlockSpec((1, G, 4), lambda b, k: (0, 0, 0)),
        ],
        out_specs=[
            pl.BlockSpec((1, G, 1), lambda b, k: (b, 0, 0)),
            pl.BlockSpec((1, G, 1), lambda b, k: (b, 0, 0)),
            pl.BlockSpec((1, 1, 1), lambda b, k: (b, 0, 0)),
        ],
        out_shape=[
            jax.ShapeDtypeStruct((B, G, 1), jnp.int32),
            jax.ShapeDtypeStruct((B, G, 1), jnp.int32),
            jax.ShapeDtypeStruct((B, 1, 1), jnp.float32),
        ],
        scratch_shapes=[
            pltpu.VMEM((G, 1), jnp.float32),
            pltpu.VMEM((G, 1), jnp.int32),
            pltpu.VMEM((G, 4), jnp.float32),
        ],
        compiler_params=pltpu.CompilerParams(
            dimension_semantics=("arbitrary", "arbitrary")),
        interpret=interpret,
    )(bpT, gt_boxes, gt_boxes)


def _stage_c(flat_idx, cls_t, li2, lbl2, bxp, interpret=False):
    return pl.pallas_call(
        _loss_body,
        in_specs=[
            pl.BlockSpec(memory_space=pltpu.SMEM),
            pl.BlockSpec(memory_space=pl.ANY),
            pl.BlockSpec((BG, 1), lambda: (0, 0)),
            pl.BlockSpec((BG, 1), lambda: (0, 0)),
            pl.BlockSpec((B, 1, 1), lambda: (0, 0, 0)),
        ],
        out_specs=pl.BlockSpec((1, 1), lambda: (0, 0)),
        out_shape=jax.ShapeDtypeStruct((1, 1), jnp.float32),
        scratch_shapes=[
            pltpu.VMEM((C, BG * 128), jnp.float32),
            pltpu.SemaphoreType.DMA,
        ],
        interpret=interpret,
    )(flat_idx, cls_t, li2, lbl2, bxp)


def kernel(cls_preds, box_preds, gt_boxes, gt_labels):
    box_t = jnp.transpose(box_preds, (0, 2, 1))
    cls_t = jnp.transpose(cls_preds, (0, 2, 1))
    flat_idx3, li3, bxp = _stage_a(box_t, gt_boxes)
    flat_idx = flat_idx3.reshape(BG)
    out = _stage_c(flat_idx, cls_t, li3.reshape(BG, 1),
                   gt_labels.reshape(BG, 1), bxp)
    return out.reshape(())
